# baseline (speedup 1.0000x reference)
import functools

import jax
import jax.numpy as jnp
from jax import lax
from jax.experimental import pallas as pl
from jax.experimental.pallas import tpu as pltpu
from jax.experimental.pallas import tpu_sc as plsc

_NC = 2
_NS = 16
_NW = _NC * _NS
_LANES = 16
_NBUF = 3

_VCHUNK = 256


@functools.partial(jax.jit, static_argnames=("v", "d"))
def _reformat(tok_t, tail2, *, v, d):
    n_chunks = v // _VCHUNK
    n_tail = tail2.shape[0]
    per_w = n_chunks // _NW
    extra = n_chunks - per_w * _NW

    mesh = plsc.VectorSubcoreMesh(core_axis_name="c", subcore_axis_name="s")

    @functools.partial(
        pl.kernel,
        out_type=jax.ShapeDtypeStruct((v * d,), jnp.float32),
        mesh=mesh,
        scratch_types=[
            [pltpu.VMEM((d, _VCHUNK), jnp.float32) for _ in range(2)],
            [pltpu.VMEM((_VCHUNK * d,), jnp.float32) for _ in range(2)],
            [pltpu.SemaphoreType.DMA for _ in range(2)],
            [pltpu.SemaphoreType.DMA for _ in range(2)],
        ],
        compiler_params=pltpu.CompilerParams(
            use_tc_tiling_on_sc=True, needs_layout_passes=False
        ),
    )
    def body(tok_hbm, tail_hbm, out_hbm, inb, outb, isem, osem):
        wid = lax.axis_index("s") * _NC + lax.axis_index("c")
        csize = _VCHUNK * d

        lanes = lax.iota(jnp.int32, _LANES)
        rowvecs = [cc * _LANES + lanes for cc in range(d // _LANES)]

        def shuffle(bsel):
            def per_v(vv, c):
                col = jnp.full((_LANES,), 0, jnp.int32) + vv
                for cc in range(d // _LANES):
                    val = plsc.load_gather(inb[bsel], [rowvecs[cc], col])
                    outb[bsel][pl.ds(vv * d + cc * _LANES, _LANES)] = val
                return c

            lax.fori_loop(0, _VCHUNK, per_v, 0, unroll=2)

        def issue_in(ci, bsel):
            return pltpu.async_copy(
                tok_hbm.at[:, pl.ds(ci * _VCHUNK, _VCHUNK)], inb[bsel],
                isem[bsel],
            )

        def issue_out(ci, bsel):
            return pltpu.async_copy(
                outb[bsel], out_hbm.at[pl.ds(ci * csize, csize)], osem[bsel]
            )

        def drain_in(bsel):
            pltpu.make_async_copy(
                tok_hbm.at[:, pl.ds(0, _VCHUNK)], inb[bsel], isem[bsel]
            ).wait()

        def drain_out(bsel):
            pltpu.make_async_copy(
                outb[bsel], out_hbm.at[pl.ds(0, csize)], osem[bsel]
            ).wait()

        issue_in(wid, 0)

        def step(g, bsel):
            gn = jnp.minimum(g + 1, per_w - 1)
            issue_in(gn * _NW + wid, 1 - bsel)
            drain_in(bsel)

            @pl.when(g >= 2)
            def _():
                drain_out(bsel)

            shuffle(bsel)
            issue_out(g * _NW + wid, bsel)

        def pair(gg, c):
            step(2 * gg, 0)
            step(2 * gg + 1, 1)
            return c

        lax.fori_loop(0, per_w // 2, pair, 0)
        if per_w % 2:
            step(per_w - 1, 0)
        drain_in(per_w % 2)
        drain_out(per_w % 2)
        drain_out(1 - per_w % 2)

        @pl.when(wid < extra)
        def _():
            ci = per_w * _NW + wid
            issue_in(ci, 0)
            drain_in(0)
            shuffle(0)
            issue_out(ci, 0)
            drain_out(0)

        if n_tail:
            @pl.when(wid == 1)
            def _():
                pltpu.async_copy(
                    tail_hbm, outb[1].at[pl.ds(0, n_tail)], isem[1]
                ).wait()
                pltpu.async_copy(
                    outb[1].at[pl.ds(0, n_tail)],
                    out_hbm.at[pl.ds(n_chunks * csize, n_tail)], osem[1],
                ).wait()

    return body(tok_t, tail2)


@functools.partial(jax.jit, static_argnames=("n_rows", "t_len", "d"))
def _emb_lookup(tok2, pos_weight, idx, *, n_rows, t_len, d):
    per_w = n_rows // _NW
    chunk = 2 * t_len
    n_chunks = per_w // chunk

    mesh = plsc.VectorSubcoreMesh(core_axis_name="c", subcore_axis_name="s")

    @functools.partial(
        pl.kernel,
        out_type=jax.ShapeDtypeStruct((n_rows, d), jnp.float32),
        mesh=mesh,
        scratch_types=[
            pltpu.VMEM((per_w,), jnp.int32),
            pltpu.VMEM((t_len, d), jnp.float32),
            [pltpu.VMEM((chunk, d), jnp.float32) for _ in range(_NBUF)],
            [pltpu.SemaphoreType.DMA for _ in range(_NBUF)],
            [pltpu.SemaphoreType.DMA for _ in range(_NBUF)],
        ],
        compiler_params=pltpu.CompilerParams(use_tc_tiling_on_sc=False),
    )
    def body(tok_hbm, pos_hbm, idx_hbm, out_hbm, idx_v, pos_v,
             rows, gsem, ssem):
        wid = lax.axis_index("s") * _NC + lax.axis_index("c")
        base = wid * per_w
        pltpu.sync_copy(idx_hbm.at[pl.ds(base, per_w)], idx_v)
        pltpu.sync_copy(pos_hbm.at[pl.ds(0, t_len)], pos_v)

        gathers = {}
        scatters = {}

        def issue_gather(g):
            b = g % _NBUF
            gathers[g] = pltpu.async_copy(
                tok_hbm.at[idx_v.at[pl.ds(g * chunk, chunk)]], rows[b], gsem[b]
            )

        def add_pos(b):
            def add_row(r, c2):
                for cc in range(d // _LANES):
                    sl = pl.ds(cc * _LANES, _LANES)
                    p = pos_v[r, sl]
                    plsc.addupdate(rows[b].at[r, sl], p)
                    plsc.addupdate(rows[b].at[r + t_len, sl], p)
                return c2

            lax.fori_loop(0, t_len, add_row, 0, unroll=2)

        issue_gather(0)
        for g in range(n_chunks):
            b = g % _NBUF
            if g + 1 < n_chunks:
                if g + 1 - _NBUF >= 0:
                    scatters.pop(g + 1 - _NBUF).wait()
                issue_gather(g + 1)
            gathers.pop(g).wait()
            add_pos(b)
            scatters[g] = pltpu.async_copy(
                rows[b], out_hbm.at[pl.ds(base + g * chunk, chunk)], ssem[b]
            )
        for g in sorted(scatters):
            scatters.pop(g).wait()

    return body(tok2, pos_weight, idx)


def kernel(x_ids, tok_weight, pos_weight):
    b, t = x_ids.shape
    v, d = tok_weight.shape
    n_rows = b * t
    assert n_rows % (_NW * 2 * t) == 0 and d % _LANES == 0 and v % 2 == 0
    idx = x_ids.reshape(-1).astype(jnp.int32)
    vlimit = (v // _VCHUNK) * _VCHUNK
    tail2 = tok_weight[vlimit:].reshape(-1)
    tok_flat = _reformat(tok_weight.T, tail2, v=v, d=d)
    tok2 = tok_flat.reshape(v, d)
    out = _emb_lookup(tok2, pos_weight, idx, n_rows=n_rows, t_len=t, d=d)
    return out.reshape(b, t, d)


# speedup vs baseline: 2.4036x; 2.4036x over previous
import functools

import jax
import jax.numpy as jnp
from jax import lax
from jax.experimental import pallas as pl
from jax.experimental.pallas import tpu as pltpu
from jax.experimental.pallas import tpu_sc as plsc

_NC = 2
_NS = 16
_NW = _NC * _NS
_LANES = 16
_NBUF = 4


@functools.partial(jax.jit, static_argnames=("n_rows", "t_len", "d"))
def _emb_lookup(tok_weight, pos_weight, idx, *, n_rows, t_len, d):
    per_w = n_rows // _NW
    chunk = 2 * t_len
    n_chunks = per_w // chunk

    mesh = plsc.VectorSubcoreMesh(core_axis_name="c", subcore_axis_name="s")

    @functools.partial(
        pl.kernel,
        out_type=jax.ShapeDtypeStruct((n_rows, 2 * d), jnp.float32),
        mesh=mesh,
        scratch_types=[
            pltpu.VMEM((per_w,), jnp.int32),
            pltpu.VMEM((t_len, d), jnp.float32),
            [pltpu.VMEM((chunk, d), jnp.float32) for _ in range(_NBUF)],
            [pltpu.SemaphoreType.DMA for _ in range(_NBUF)],
            [pltpu.SemaphoreType.DMA for _ in range(_NBUF)],
        ],
        compiler_params=pltpu.CompilerParams(use_tc_tiling_on_sc=False),
    )
    def body(tok_hbm, pos_hbm, idx_hbm, out_hbm, idx_v, pos_v, rows, gsem, ssem):
        wid = lax.axis_index("s") * _NC + lax.axis_index("c")
        base = wid * per_w
        pltpu.sync_copy(idx_hbm.at[pl.ds(base, per_w)], idx_v)
        pltpu.sync_copy(pos_hbm.at[pl.ds(0, t_len)], pos_v)

        gathers = {}
        scatters = {}

        def issue_gather(g):
            b = g % _NBUF
            gathers[g] = pltpu.async_copy(
                tok_hbm.at[idx_v.at[pl.ds(g * chunk, chunk)]], rows[b], gsem[b]
            )

        def add_pos(b):
            def add_row(r, c2):
                for cc in range(d // _LANES):
                    sl = pl.ds(cc * _LANES, _LANES)
                    p = pos_v[r, sl]
                    plsc.addupdate(rows[b].at[r, sl], p)
                    plsc.addupdate(rows[b].at[r + t_len, sl], p)
                return c2

            lax.fori_loop(0, t_len, add_row, 0, unroll=2)

        issue_gather(0)
        if n_chunks > 1:
            issue_gather(1)
        for g in range(n_chunks):
            b = g % _NBUF
            if g + 2 < n_chunks:
                nb = (g + 2) % _NBUF
                if g + 2 - _NBUF >= 0:
                    scatters.pop(g + 2 - _NBUF).wait()
                issue_gather(g + 2)
            gathers.pop(g).wait()
            add_pos(b)
            scatters[g] = pltpu.async_copy(
                rows[b],
                out_hbm.at[pl.ds(base + g * chunk, chunk), pl.ds(0, d)],
                ssem[b],
            )
        for g in sorted(scatters):
            scatters.pop(g).wait()

    return body(tok_weight, pos_weight, idx)


def kernel(x_ids, tok_weight, pos_weight):
    b, t = x_ids.shape
    d = tok_weight.shape[1]
    n_rows = b * t
    assert n_rows % (_NW * 2 * t) == 0 and d % _LANES == 0
    idx = x_ids.reshape(-1).astype(jnp.int32)
    out2 = _emb_lookup(tok_weight, pos_weight, idx, n_rows=n_rows, t_len=t, d=d)
    return out2[:, :d].reshape(b, t, d)
